# baseline (speedup 1.0000x reference)
import jax
import jax.numpy as jnp
from jax import lax
from jax.experimental import pallas as pl
from jax.experimental.pallas import tpu as pltpu
from jax.experimental.pallas import tpu_sc as plsc

_DIM = 768
_ML = 1000
_S = 512
_NC = 2
_NW = 32
_IPW = _S // _NW
_RV = 1024
_RPT = _RV // 16
_ROWW = _S * _DIM


def _sc_body(tab_hbm, out_hbm, stage, rev, ssem, dsem):
    cid = lax.axis_index("c")
    sid = lax.axis_index("s")
    wid = sid * _NC + cid
    i0 = wid * _IPW

    cp = pltpu.make_async_copy(
        tab_hbm.at[pl.ds(488 + sid * _RPT, _RPT), :], stage, ssem
    )
    cp.start()
    cp.wait()

    def _put_row(r, carry):
        m = 1023 - _RPT * sid - r
        pltpu.make_async_copy(
            stage.at[r], rev.at[pl.ds(m * _DIM, _DIM)], ssem
        ).start()
        return carry

    lax.fori_loop(0, _RPT, _put_row, 0, unroll=False)

    def _drain_row(r, carry):
        pltpu.make_async_copy(
            stage.at[0], rev.at[pl.ds(0, _DIM)], ssem
        ).wait()
        return carry

    lax.fori_loop(0, _RPT, _drain_row, 0, unroll=False)
    plsc.subcore_barrier()

    for t in range(_IPW):
        i = i0 + t
        pltpu.make_async_copy(
            rev.at[pl.ds((_S - 1 - i) * _DIM, _ROWW)],
            out_hbm.at[pl.ds(i * _ROWW, _ROWW)],
            dsem,
        ).start()
    for t in range(_IPW):
        pltpu.make_async_copy(
            rev.at[pl.ds(0, _ROWW)], out_hbm.at[pl.ds(0, _ROWW)], dsem
        ).wait()


def kernel(hidden_states, pe_k_weight):
    s = hidden_states.shape[1]
    mesh = plsc.VectorSubcoreMesh(core_axis_name="c", subcore_axis_name="s")
    run = pl.kernel(
        _sc_body,
        out_type=jax.ShapeDtypeStruct((s * s * _DIM,), jnp.float32),
        mesh=mesh,
        scratch_types=[
            pltpu.VMEM((_RPT, _DIM), jnp.float32),
            pltpu.VMEM_SHARED((_RV * _DIM,), jnp.float32),
            pltpu.SemaphoreType.DMA,
            pltpu.SemaphoreType.DMA,
        ],
    )
    return run(pe_k_weight).reshape(s, s, _DIM)


# speedup vs baseline: 3.1743x; 3.1743x over previous
import jax
import jax.numpy as jnp
from jax import lax
from jax.experimental import pallas as pl
from jax.experimental.pallas import tpu as pltpu
from jax.experimental.pallas import tpu_sc as plsc

_DIM = 768
_ML = 1000
_S = 512
_NC = 2
_NW = 32
_CH = 64
_NCK = _S // _CH
_G = 8
_GR = 128


def _sc_body(tab_hbm, out_hbm, idx, buf, gsem, ssem):
    wid = lax.axis_index("s") * _NC + lax.axis_index("c")

    def _step(it, carry):
        g = 2 * wid + it // _NCK
        k = it % _NCK
        beta = 64 * (g // 8) + g % 8
        j0 = pl.multiple_of(_CH * k, _CH)
        base = beta + 56 + _ML - j0

        @pl.when(it > 0)
        def _():
            for m in range(_G):
                pltpu.make_async_copy(
                    buf.at[pl.ds(0, _CH), :],
                    out_hbm.at[0, pl.ds(0, _CH), :],
                    ssem,
                ).wait()

        for q in range(_GR // 16):
            idx[pl.ds(16 * q, 16)] = base - 16 * q - lax.iota(jnp.int32, 16)
        cp = pltpu.make_async_copy(tab_hbm.at[idx], buf, gsem)
        cp.start()
        cp.wait()

        for m in range(_G):
            pltpu.make_async_copy(
                buf.at[pl.ds(56 - 8 * m, _CH), :],
                out_hbm.at[beta + 8 * m, pl.ds(j0, _CH), :],
                ssem,
            ).start()
        return carry

    lax.fori_loop(0, 2 * _NCK, _step, 0, unroll=False)

    for m in range(_G):
        pltpu.make_async_copy(
            buf.at[pl.ds(0, _CH), :], out_hbm.at[0, pl.ds(0, _CH), :], ssem
        ).wait()


def kernel(hidden_states, pe_k_weight):
    s = hidden_states.shape[1]
    mesh = plsc.VectorSubcoreMesh(core_axis_name="c", subcore_axis_name="s")
    run = pl.kernel(
        _sc_body,
        out_type=jax.ShapeDtypeStruct((s, s, _DIM), jnp.float32),
        mesh=mesh,
        scratch_types=[
            pltpu.VMEM((_GR,), jnp.int32),
            pltpu.VMEM((_GR, _DIM), jnp.float32),
            pltpu.SemaphoreType.DMA,
            pltpu.SemaphoreType.DMA,
        ],
    )
    return run(pe_k_weight)


# speedup vs baseline: 3.2083x; 1.0107x over previous
import jax
import jax.numpy as jnp
from jax import lax
from jax.experimental import pallas as pl
from jax.experimental.pallas import tpu as pltpu
from jax.experimental.pallas import tpu_sc as plsc

_DIM = 768
_ML = 1000
_S = 512
_NC = 2
_NW = 32
_CH = 64
_NCK = _S // _CH
_G = 8
_GR = 128
_UPW = 15
_LPW = 1
_RV = 640
_T0 = 480


def _sc_body(tab_hbm, out_hbm, idx_s, idx, buf, rev, gsem, ssem, lsem):
    cid = lax.axis_index("c")
    sid = lax.axis_index("s")
    wid = sid * _NC + cid

    @pl.when(sid < _RV // 64)
    def _stage():
        base_s = 1031 - 4 * cid - 64 * sid
        for q in range(4):
            idx_s[pl.ds(16 * q, 16)] = base_s - 16 * q - lax.iota(jnp.int32, 16)
        cp = pltpu.make_async_copy(
            tab_hbm.at[idx_s], buf.at[pl.ds(0, 64), :], gsem
        )
        cp.start()
        cp.wait()
        pltpu.sync_copy(buf.at[pl.ds(0, 64), :], rev.at[pl.ds(64 * sid, 64), :])

    plsc.subcore_barrier()

    for l in range(_LPW):
        q = 48 + _LPW * sid + l
        i = 8 * q + 7 - 4 * cid
        m0 = pl.multiple_of(8 * (63 - q), 8)
        pltpu.make_async_copy(
            rev.at[pl.ds(m0, _S), :], out_hbm.at[i], lsem
        ).start()

    def _unit(s, carry):
        unit = _UPW * wid + s
        grp = unit // _NCK
        k = unit % _NCK
        r_idx = grp // 8
        rr = jnp.where(
            grp < 48,
            r_idx + jnp.where(r_idx >= 3, 1, 0),
            jnp.where(grp < 54, 3, 7),
        )
        u = jnp.where(grp < 48, grp % 8, jnp.where(grp < 54, grp - 48, grp - 54))
        j0 = pl.multiple_of(_CH * k, _CH)
        base = 64 * u + 56 + rr + _ML - _T0 - j0

        @pl.when(s > 0)
        def _():
            for m in range(_G):
                pltpu.make_async_copy(
                    buf.at[pl.ds(0, _CH), :],
                    out_hbm.at[0, pl.ds(0, _CH), :],
                    ssem,
                ).wait()

        for q in range(_GR // 16):
            idx[pl.ds(16 * q, 16)] = base - 16 * q - lax.iota(jnp.int32, 16)
        cp = pltpu.make_async_copy(tab_hbm.at[idx], buf, gsem)
        cp.start()
        cp.wait()

        for m in range(_G):
            pltpu.make_async_copy(
                buf.at[pl.ds(56 - 8 * m, _CH), :],
                out_hbm.at[64 * u + 8 * m + rr, pl.ds(j0, _CH), :],
                ssem,
            ).start()
        return carry

    lax.fori_loop(0, _UPW, _unit, 0, unroll=False)

    for m in range(_G):
        pltpu.make_async_copy(
            buf.at[pl.ds(0, _CH), :], out_hbm.at[0, pl.ds(0, _CH), :], ssem
        ).wait()
    for l in range(_LPW):
        pltpu.make_async_copy(
            rev.at[pl.ds(0, _S), :], out_hbm.at[0], lsem
        ).wait()


def kernel(hidden_states, pe_k_weight):
    s = hidden_states.shape[1]
    mesh = plsc.VectorSubcoreMesh(core_axis_name="c", subcore_axis_name="s")
    run = pl.kernel(
        _sc_body,
        out_type=jax.ShapeDtypeStruct((s, s, _DIM), jnp.float32),
        mesh=mesh,
        scratch_types=[
            pltpu.VMEM((64,), jnp.int32),
            pltpu.VMEM((_GR,), jnp.int32),
            pltpu.VMEM((_GR, _DIM), jnp.float32),
            pltpu.VMEM_SHARED((_RV, _DIM), jnp.float32),
            pltpu.SemaphoreType.DMA,
            pltpu.SemaphoreType.DMA,
            pltpu.SemaphoreType.DMA,
        ],
    )
    return run(pe_k_weight[_T0:_T0 + 1032])
